# Initial kernel scaffold; baseline (speedup 1.0000x reference)
import jax
import jax.numpy as jnp
from jax.experimental import pallas as pl


def kernel(actors_feats, actor_idcs, actor_ctrs, graph_ctrs, graph_feats, graph_idcs, graph_turn, graph_control, graph_intersect, pre_u, pre_v, suc_u, suc_v, left_u, left_v, right_u, right_v, Wa1, ba1, Wa2, ba2, Wm_in, bm_in, Wf_ctr, Wf_pre, Wf_suc, Wf_left, Wf_right, Wg_ctr, Wg_pre, Wg_suc, Wg_left, Wg_right, bf, bg, Wmeta, bmeta, Wq_a2m, Wk_a2m, Wv_a2m, Wo_a2m, Wq_m2a, Wk_m2a, Wv_m2a, Wo_m2a, Wq_a2a, Wk_a2a, Wv_a2a, Wo_a2a, Wh1, bh1, Wreg, breg, Wd2, Wcls, bcls):
    raise NotImplementedError("write your pallas kernel here")


# speedup vs baseline: 2.0748x; 2.0748x over previous
import functools

import jax
import jax.numpy as jnp
from jax import lax
from jax.experimental import pallas as pl
from jax.experimental.pallas import tpu as pltpu
from jax.experimental.pallas import tpu_sc as plsc

_NA, _NN, _D = 2048, 10000, 256
_NNP = 10240
_CHUNK = 128
_EMULT = 16 * _CHUNK
_F32 = jnp.float32


def _ln(y):
    mu = jnp.mean(y, axis=-1, keepdims=True)
    var = jnp.mean((y - mu) ** 2, axis=-1, keepdims=True)
    return (y - mu) * lax.rsqrt(var + 1e-5)


def _mm(xs, w, b=None, res=None, res2=None, mode="none", bn=256):
    n = xs[0].shape[0]
    ks = [x.shape[1] for x in xs]
    m = w.shape[1]
    nx = len(xs)
    have_b, have_r, have_r2 = b is not None, res is not None, res2 is not None

    def body(*refs):
        xrefs = refs[:nx]
        w_ref = refs[nx]
        idx = nx + 1
        b_ref = r_ref = r2_ref = None
        if have_b:
            b_ref = refs[idx]; idx += 1
        if have_r:
            r_ref = refs[idx]; idx += 1
        if have_r2:
            r2_ref = refs[idx]; idx += 1
        out_ref = refs[idx]
        off = 0
        y = None
        for xr, k in zip(xrefs, ks):
            part = jnp.dot(xr[...].astype(jnp.bfloat16),
                           w_ref[off:off + k, :].astype(jnp.bfloat16),
                           preferred_element_type=_F32)
            y = part if y is None else y + part
            off += k
        if have_b:
            y = y + b_ref[...]
        if mode == "ln_relu":
            y = jnp.maximum(_ln(y), 0.0)
        if have_r:
            y = y + r_ref[...]
        if have_r2:
            y = y + r2_ref[...]
        out_ref[...] = y

    in_specs = [pl.BlockSpec((bn, k), lambda i: (i, 0)) for k in ks]
    in_specs.append(pl.BlockSpec((sum(ks), m), lambda i: (0, 0)))
    args = list(xs) + [w]
    if have_b:
        in_specs.append(pl.BlockSpec((1, m), lambda i: (0, 0)))
        args.append(b.reshape(1, m))
    if have_r:
        in_specs.append(pl.BlockSpec((bn, m), lambda i: (i, 0)))
        args.append(res)
    if have_r2:
        in_specs.append(pl.BlockSpec((bn, m), lambda i: (i, 0)))
        args.append(res2)
    return pl.pallas_call(
        body,
        grid=(n // bn,),
        in_specs=in_specs,
        out_specs=pl.BlockSpec((bn, m), lambda i: (i, 0)),
        out_shape=jax.ShapeDtypeStruct((n, m), _F32),
    )(*args)


def _attn(qa, ka, v, cd, cst, bd=256, bs=2048):
    ndp, kw = qa.shape
    nsp = ka.shape[0]
    nsb = nsp // bs

    def score(q_ref, k_ref, cd_ref, cst_ref):
        s = lax.dot_general(q_ref[...].astype(jnp.bfloat16),
                            k_ref[...].astype(jnp.bfloat16),
                            (((1,), (1,)), ((), ())),
                            preferred_element_type=_F32)
        s2 = jnp.dot(cd_ref[...].astype(jnp.bfloat16),
                     cst_ref[...].astype(jnp.bfloat16),
                     preferred_element_type=_F32)
        return s + 0.2 * s2 + cst_ref[2:3, :] + cst_ref[3:4, :]

    def body1(q_ref, k_ref, v_ref, cd_ref, cst_ref, o_ref):
        s = score(q_ref, k_ref, cd_ref, cst_ref)
        p = jnp.exp(s - jnp.max(s, axis=1, keepdims=True))
        w = p / jnp.sum(p, axis=1, keepdims=True)
        o_ref[...] = jnp.dot(w.astype(jnp.bfloat16),
                             v_ref[...].astype(jnp.bfloat16),
                             preferred_element_type=_F32)

    def body(q_ref, k_ref, v_ref, cd_ref, cst_ref, o_ref, acc, mrow, lrow):
        j = pl.program_id(1)

        @pl.when(j == 0)
        def _():
            acc[...] = jnp.zeros_like(acc)
            mrow[...] = jnp.full_like(mrow, -1e30)
            lrow[...] = jnp.zeros_like(lrow)

        s = score(q_ref, k_ref, cd_ref, cst_ref)
        mj = jnp.max(s, axis=1, keepdims=True)
        mold = mrow[:, 0:1]
        lold = lrow[:, 0:1]
        mnew = jnp.maximum(mold, mj)
        alpha = jnp.exp(mold - mnew)
        p = jnp.exp(s - mnew)
        lnew = lold * alpha + jnp.sum(p, axis=1, keepdims=True)
        accnew = acc[...] * alpha + jnp.dot(p.astype(jnp.bfloat16),
                                            v_ref[...].astype(jnp.bfloat16),
                                            preferred_element_type=_F32)
        mrow[...] = jnp.broadcast_to(mnew, mrow.shape)
        lrow[...] = jnp.broadcast_to(lnew, lrow.shape)
        acc[...] = accnew

        @pl.when(j == nsb - 1)
        def _():
            o_ref[...] = accnew / lnew

    return pl.pallas_call(
        body1 if nsb == 1 else body,
        grid=(ndp // bd, nsb),
        in_specs=[
            pl.BlockSpec((bd, kw), lambda i, j: (i, 0)),
            pl.BlockSpec((bs, kw), lambda i, j: (j, 0)),
            pl.BlockSpec((bs, _D), lambda i, j: (j, 0)),
            pl.BlockSpec((bd, 8), lambda i, j: (i, 0)),
            pl.BlockSpec((8, bs), lambda i, j: (0, j)),
        ],
        out_specs=pl.BlockSpec((bd, _D), lambda i, j: (i, 0)),
        out_shape=jax.ShapeDtypeStruct((ndp, _D), _F32),
        scratch_shapes=[] if nsb == 1 else [
            pltpu.VMEM((bd, _D), _F32),
            pltpu.VMEM((bd, 128), _F32),
            pltpu.VMEM((bd, 128), _F32),
        ],
    )(qa, ka, v, cd, cst)


def _cls_head(h, dd48, wd2p, wclsp, bpad, bn=256):
    def body(h_ref, d_ref, wd_ref, wc_ref, b_ref, o_ref):
        hblk = h_ref[...]
        wd = wd_ref[...].astype(jnp.bfloat16)
        wc = wc_ref[...].astype(jnp.bfloat16)
        for k in range(6):
            y = jnp.dot(d_ref[:, 8 * k:8 * k + 8].astype(jnp.bfloat16), wd,
                        preferred_element_type=_F32) + hblk
            z = jnp.maximum(_ln(y), 0.0)
            o_ref[:, 128 * k:128 * k + 128] = (
                jnp.dot(z.astype(jnp.bfloat16), wc,
                        preferred_element_type=_F32)
                + b_ref[...])

    return pl.pallas_call(
        body,
        grid=(_NA // bn,),
        in_specs=[
            pl.BlockSpec((bn, _D), lambda i: (i, 0)),
            pl.BlockSpec((bn, 48), lambda i: (i, 0)),
            pl.BlockSpec((8, _D), lambda i: (0, 0)),
            pl.BlockSpec((_D, 128), lambda i: (0, 0)),
            pl.BlockSpec((1, 128), lambda i: (0, 0)),
        ],
        out_specs=pl.BlockSpec((bn, 768), lambda i: (i, 0)),
        out_shape=jax.ShapeDtypeStruct((_NA, 768), _F32),
    )(h, dd48, wd2p, wclsp, bpad)


def _edge_agg_sc(mflat, u_pad, v_pad, zrows):
    epad = u_pad.shape[0]
    epsub = epad // 16
    nchunks = epsub // _CHUNK
    rows_per_sub = _NNP // 16

    mesh = plsc.VectorSubcoreMesh(core_axis_name="c", subcore_axis_name="s")

    @functools.partial(
        pl.kernel,
        out_type=jax.ShapeDtypeStruct((_NNP, _D), _F32),
        mesh=mesh,
        scratch_types=[
            pltpu.VMEM((_CHUNK,), jnp.int32),
            pltpu.VMEM((_CHUNK,), jnp.int32),
            pltpu.VMEM((_CHUNK,), jnp.int32),
            pltpu.VMEM((_CHUNK, 128), _F32),
            pltpu.VMEM_SHARED((_NNP, 128), _F32),
        ],
    )
    def agg_kernel(mflat_hbm, u_hbm, v_hbm, z_hbm, out_hbm,
                   uv, uadj, vv, rows, acc):
        c = lax.axis_index("c")
        s = lax.axis_index("s")
        coff = c * _NNP
        pltpu.sync_copy(z_hbm, acc.at[pl.ds(s * rows_per_sub, rows_per_sub)])
        plsc.subcore_barrier()

        def chunk(i, carry):
            base = s * epsub + i * _CHUNK
            pltpu.sync_copy(u_hbm.at[pl.ds(base, _CHUNK)], uv)
            pltpu.sync_copy(v_hbm.at[pl.ds(base, _CHUNK)], vv)
            for t in range(_CHUNK // 16):
                uadj[pl.ds(16 * t, 16)] = uv[pl.ds(16 * t, 16)] + coff
            pltpu.sync_copy(mflat_hbm.at[uadj], rows)
            pltpu.sync_copy(rows, acc.at[vv], add=True)
            return carry

        lax.fori_loop(0, nchunks, chunk, 0)
        plsc.subcore_barrier()
        pltpu.sync_copy(
            acc.at[pl.ds(s * rows_per_sub, rows_per_sub)],
            out_hbm.at[pl.ds(s * rows_per_sub, rows_per_sub),
                       pl.ds(c * 128, 128)])

    return agg_kernel(mflat, u_pad, v_pad, zrows)


def kernel(actors_feats, actor_idcs, actor_ctrs, graph_ctrs, graph_feats,
           graph_idcs, graph_turn, graph_control, graph_intersect,
           pre_u, pre_v, suc_u, suc_v, left_u, left_v, right_u, right_v,
           Wa1, ba1, Wa2, ba2, Wm_in, bm_in,
           Wf_ctr, Wf_pre, Wf_suc, Wf_left, Wf_right,
           Wg_ctr, Wg_pre, Wg_suc, Wg_left, Wg_right,
           bf, bg, Wmeta, bmeta,
           Wq_a2m, Wk_a2m, Wv_a2m, Wo_a2m,
           Wq_m2a, Wk_m2a, Wv_m2a, Wo_m2a,
           Wq_a2a, Wk_a2a, Wv_a2a, Wo_a2a,
           Wh1, bh1, Wreg, breg, Wd2, Wcls, bcls):
    npad = _NNP - _NN

    af = jnp.pad(actors_feats.reshape(_NA, 60), ((0, 0), (0, 4)))
    a0 = _mm([af], jnp.pad(Wa1, ((0, 4), (0, 0))), ba1, mode="ln_relu")
    a = _mm([a0], Wa2, ba2, res=a0, mode="ln_relu")

    enc = jnp.pad(jnp.concatenate([graph_ctrs, graph_feats], -1),
                  ((0, npad), (0, 4)))
    m0 = _mm([enc], jnp.pad(Wm_in, ((0, 4), (0, 0))), bm_in, mode="ln_relu")
    metain = jnp.pad(
        jnp.concatenate([graph_turn, graph_control[:, None],
                         graph_intersect[:, None]], -1),
        ((0, npad), (0, 4)))
    meta = _mm([metain], jnp.pad(Wmeta, ((0, 4), (0, 0))), bmeta)

    def pad_edges(u, v):
        e = u.shape[0]
        ep = ((e + _EMULT - 1) // _EMULT) * _EMULT
        return (jnp.pad(u, (0, ep - e)),
                jnp.pad(v, (0, ep - e), constant_values=_NN))

    edge_lists = [pad_edges(pre_u, pre_v), pad_edges(suc_u, suc_v),
                  pad_edges(left_u, left_v), pad_edges(right_u, right_v)]
    zrows = jnp.zeros((_NNP // 16, 128), _F32)

    def edge_aggs(m):
        mflat = jnp.concatenate([m[:, :128], m[:, 128:]], axis=0)
        return [_edge_agg_sc(mflat, u, v, zrows) for (u, v) in edge_lists]

    aggs = edge_aggs(m0)
    Wf = jnp.concatenate([Wf_ctr, Wf_pre, Wf_suc, Wf_left, Wf_right], axis=0)
    m1 = _mm([m0] + aggs, Wf, bf, res=m0, res2=meta, mode="ln_relu")

    gctr = jnp.pad(graph_ctrs, ((0, npad), (0, 0)))

    def attention(x_dst, ctr_d, x_src, ctr_s, n_src_real, Wq, Wk, Wv, Wo):
        nd = x_dst.shape[0]
        ns = x_src.shape[0]
        q = _mm([x_dst], Wq * (1.0 / 16.0))
        kv = _mm([x_src], jnp.concatenate([Wk, Wv], axis=1))
        k, v = kv[:, :_D], kv[:, _D:]
        maskrow = jnp.where(jnp.arange(ns) < n_src_real, 0.0, -1e9)[None, :]
        cd = jnp.pad(ctr_d, ((0, 0), (0, 6)))
        cst = jnp.concatenate(
            [ctr_s.T, -0.1 * jnp.sum(ctr_s ** 2, -1)[None, :],
             maskrow, jnp.zeros((4, ns), _F32)], axis=0)
        ctx = _attn(q, k, v, cd, cst)
        return _mm([ctx], Wo, res=x_dst)

    m2 = attention(m1, gctr, a, actor_ctrs, _NA,
                   Wq_a2m, Wk_a2m, Wv_a2m, Wo_a2m)

    aggs2 = edge_aggs(m2)
    Wg = jnp.concatenate([Wg_ctr, Wg_pre, Wg_suc, Wg_left, Wg_right], axis=0)
    m3 = _mm([m2] + aggs2, Wg, bg, res=m2, mode="ln_relu")

    a2 = attention(a, actor_ctrs, m3, gctr, _NN,
                   Wq_m2a, Wk_m2a, Wv_m2a, Wo_m2a)
    a3 = attention(a2, actor_ctrs, a2, actor_ctrs, _NA,
                   Wq_a2a, Wk_a2a, Wv_a2a, Wo_a2a)

    h = _mm([a3], Wh1, bh1, mode="ln_relu")
    ctr_tiled = jnp.pad(jnp.tile(actor_ctrs, (1, 180)), ((0, 0), (0, 24)))
    regf = _mm([h], jnp.pad(Wreg, ((0, 0), (0, 24))),
               jnp.pad(breg, (0, 24)), res=ctr_tiled)
    reg = regf[:, :360].reshape(_NA, 6, 30, 2)
    dest = reg[:, :, -1, :]
    dd48 = jnp.pad(dest - actor_ctrs[:, None, :],
                   ((0, 0), (0, 0), (0, 6))).reshape(_NA, 48)
    clsout = _cls_head(h, dd48,
                       jnp.pad(Wd2, ((0, 6), (0, 0))),
                       jnp.pad(Wcls, ((0, 0), (0, 127))),
                       jnp.broadcast_to(bcls.reshape(1, 1), (1, 128)))
    cls = clsout.reshape(_NA, 6, 128)[:, :, 0]
    return reg, cls


# speedup vs baseline: 2.0995x; 1.0119x over previous
import functools

import jax
import jax.numpy as jnp
from jax import lax
from jax.experimental import pallas as pl
from jax.experimental.pallas import tpu as pltpu
from jax.experimental.pallas import tpu_sc as plsc

_NA, _NN, _D = 2048, 10000, 256
_NNP = 10240
_CHUNK = 64
_NBUF = 4
_EMULT = 16 * _CHUNK * _NBUF
_F32 = jnp.float32


def _ln(y):
    mu = jnp.mean(y, axis=-1, keepdims=True)
    var = jnp.mean((y - mu) ** 2, axis=-1, keepdims=True)
    return (y - mu) * lax.rsqrt(var + 1e-5)


def _mm(xs, w, b=None, res=None, res2=None, mode="none", bn=256):
    n = xs[0].shape[0]
    ks = [x.shape[1] for x in xs]
    m = w.shape[1]
    nx = len(xs)
    have_b, have_r, have_r2 = b is not None, res is not None, res2 is not None

    def body(*refs):
        xrefs = refs[:nx]
        w_ref = refs[nx]
        idx = nx + 1
        b_ref = r_ref = r2_ref = None
        if have_b:
            b_ref = refs[idx]; idx += 1
        if have_r:
            r_ref = refs[idx]; idx += 1
        if have_r2:
            r2_ref = refs[idx]; idx += 1
        out_ref = refs[idx]
        off = 0
        y = None
        for xr, k in zip(xrefs, ks):
            part = jnp.dot(xr[...].astype(jnp.bfloat16),
                           w_ref[off:off + k, :].astype(jnp.bfloat16),
                           preferred_element_type=_F32)
            y = part if y is None else y + part
            off += k
        if have_b:
            y = y + b_ref[...]
        if mode == "ln_relu":
            y = jnp.maximum(_ln(y), 0.0)
        if have_r:
            y = y + r_ref[...]
        if have_r2:
            y = y + r2_ref[...]
        out_ref[...] = y

    in_specs = [pl.BlockSpec((bn, k), lambda i: (i, 0)) for k in ks]
    in_specs.append(pl.BlockSpec((sum(ks), m), lambda i: (0, 0)))
    args = list(xs) + [w]
    if have_b:
        in_specs.append(pl.BlockSpec((1, m), lambda i: (0, 0)))
        args.append(b.reshape(1, m))
    if have_r:
        in_specs.append(pl.BlockSpec((bn, m), lambda i: (i, 0)))
        args.append(res)
    if have_r2:
        in_specs.append(pl.BlockSpec((bn, m), lambda i: (i, 0)))
        args.append(res2)
    return pl.pallas_call(
        body,
        grid=(n // bn,),
        in_specs=in_specs,
        out_specs=pl.BlockSpec((bn, m), lambda i: (i, 0)),
        out_shape=jax.ShapeDtypeStruct((n, m), _F32),
    )(*args)


def _attn(qa, ka, v, cd, cst, bd=256, bs=2048):
    ndp, kw = qa.shape
    nsp = ka.shape[0]
    nsb = nsp // bs

    def score(q_ref, k_ref, cd_ref, cst_ref):
        s = lax.dot_general(q_ref[...].astype(jnp.bfloat16),
                            k_ref[...].astype(jnp.bfloat16),
                            (((1,), (1,)), ((), ())),
                            preferred_element_type=_F32)
        s2 = jnp.dot(cd_ref[...].astype(jnp.bfloat16),
                     cst_ref[...].astype(jnp.bfloat16),
                     preferred_element_type=_F32)
        return s + 0.2 * s2 + cst_ref[2:3, :] + cst_ref[3:4, :]

    def body1(q_ref, k_ref, v_ref, cd_ref, cst_ref, o_ref):
        s = score(q_ref, k_ref, cd_ref, cst_ref)
        p = jnp.exp(s - jnp.max(s, axis=1, keepdims=True))
        w = p / jnp.sum(p, axis=1, keepdims=True)
        o_ref[...] = jnp.dot(w.astype(jnp.bfloat16),
                             v_ref[...].astype(jnp.bfloat16),
                             preferred_element_type=_F32)

    def body(q_ref, k_ref, v_ref, cd_ref, cst_ref, o_ref, acc, mrow, lrow):
        j = pl.program_id(1)

        @pl.when(j == 0)
        def _():
            acc[...] = jnp.zeros_like(acc)
            mrow[...] = jnp.full_like(mrow, -1e30)
            lrow[...] = jnp.zeros_like(lrow)

        s = score(q_ref, k_ref, cd_ref, cst_ref)
        mj = jnp.max(s, axis=1, keepdims=True)
        mold = mrow[:, 0:1]
        lold = lrow[:, 0:1]
        mnew = jnp.maximum(mold, mj)
        alpha = jnp.exp(mold - mnew)
        p = jnp.exp(s - mnew)
        lnew = lold * alpha + jnp.sum(p, axis=1, keepdims=True)
        accnew = acc[...] * alpha + jnp.dot(p.astype(jnp.bfloat16),
                                            v_ref[...].astype(jnp.bfloat16),
                                            preferred_element_type=_F32)
        mrow[...] = jnp.broadcast_to(mnew, mrow.shape)
        lrow[...] = jnp.broadcast_to(lnew, lrow.shape)
        acc[...] = accnew

        @pl.when(j == nsb - 1)
        def _():
            o_ref[...] = accnew / lnew

    return pl.pallas_call(
        body1 if nsb == 1 else body,
        grid=(ndp // bd, nsb),
        in_specs=[
            pl.BlockSpec((bd, kw), lambda i, j: (i, 0)),
            pl.BlockSpec((bs, kw), lambda i, j: (j, 0)),
            pl.BlockSpec((bs, _D), lambda i, j: (j, 0)),
            pl.BlockSpec((bd, 8), lambda i, j: (i, 0)),
            pl.BlockSpec((8, bs), lambda i, j: (0, j)),
        ],
        out_specs=pl.BlockSpec((bd, _D), lambda i, j: (i, 0)),
        out_shape=jax.ShapeDtypeStruct((ndp, _D), _F32),
        scratch_shapes=[] if nsb == 1 else [
            pltpu.VMEM((bd, _D), _F32),
            pltpu.VMEM((bd, 128), _F32),
            pltpu.VMEM((bd, 128), _F32),
        ],
    )(qa, ka, v, cd, cst)


def _cls_head(h, dd48, wd2p, wclsp, bpad, bn=256):
    def body(h_ref, d_ref, wd_ref, wc_ref, b_ref, o_ref):
        hblk = h_ref[...]
        wd = wd_ref[...].astype(jnp.bfloat16)
        wc = wc_ref[...].astype(jnp.bfloat16)
        for k in range(6):
            y = jnp.dot(d_ref[:, 8 * k:8 * k + 8].astype(jnp.bfloat16), wd,
                        preferred_element_type=_F32) + hblk
            z = jnp.maximum(_ln(y), 0.0)
            o_ref[:, 128 * k:128 * k + 128] = (
                jnp.dot(z.astype(jnp.bfloat16), wc,
                        preferred_element_type=_F32)
                + b_ref[...])

    return pl.pallas_call(
        body,
        grid=(_NA // bn,),
        in_specs=[
            pl.BlockSpec((bn, _D), lambda i: (i, 0)),
            pl.BlockSpec((bn, 48), lambda i: (i, 0)),
            pl.BlockSpec((8, _D), lambda i: (0, 0)),
            pl.BlockSpec((_D, 128), lambda i: (0, 0)),
            pl.BlockSpec((1, 128), lambda i: (0, 0)),
        ],
        out_specs=pl.BlockSpec((bn, 768), lambda i: (i, 0)),
        out_shape=jax.ShapeDtypeStruct((_NA, 768), _F32),
    )(h, dd48, wd2p, wclsp, bpad)


def _edge_agg_sc(mflat, u_pad, v_pad, zrows):
    epad = u_pad.shape[0]
    epsub = epad // 16
    nsuper = epsub // (_CHUNK * _NBUF)
    rows_per_sub = _NNP // 16

    mesh = plsc.VectorSubcoreMesh(core_axis_name="c", subcore_axis_name="s")

    @functools.partial(
        pl.kernel,
        out_type=jax.ShapeDtypeStruct((_NNP, _D), _F32),
        mesh=mesh,
        scratch_types=[
            [pltpu.VMEM((_CHUNK,), jnp.int32) for _ in range(_NBUF)],
            [pltpu.VMEM((_CHUNK,), jnp.int32) for _ in range(_NBUF)],
            [pltpu.VMEM((_CHUNK,), jnp.int32) for _ in range(_NBUF)],
            [pltpu.VMEM((_CHUNK, 128), _F32) for _ in range(_NBUF)],
            [pltpu.SemaphoreType.DMA for _ in range(4 * _NBUF)],
            pltpu.VMEM_SHARED((_NNP, 128), _F32),
        ],
    )
    def agg_kernel(mflat_hbm, u_hbm, v_hbm, z_hbm, out_hbm,
                   ubufs, vbufs, abufs, rowbufs, sems, acc):
        c = lax.axis_index("c")
        s = lax.axis_index("s")
        coff = c * _NNP
        su, sv = sems[:_NBUF], sems[_NBUF:2 * _NBUF]
        sg, ss = sems[2 * _NBUF:3 * _NBUF], sems[3 * _NBUF:]
        pltpu.sync_copy(z_hbm, acc.at[pl.ds(s * rows_per_sub, rows_per_sub)])
        plsc.subcore_barrier()

        def superchunk(j, carry):
            base = s * epsub + j * (_CHUNK * _NBUF)
            du, dv, dg, dsc = [], [], [], []
            for b in range(_NBUF):
                du.append(pltpu.async_copy(
                    u_hbm.at[pl.ds(base + b * _CHUNK, _CHUNK)], ubufs[b],
                    su[b]))
                dv.append(pltpu.async_copy(
                    v_hbm.at[pl.ds(base + b * _CHUNK, _CHUNK)], vbufs[b],
                    sv[b]))
            for b in range(_NBUF):
                du[b].wait()
                for t in range(_CHUNK // 16):
                    abufs[b][pl.ds(16 * t, 16)] = (
                        ubufs[b][pl.ds(16 * t, 16)] + coff)
                dg.append(pltpu.async_copy(
                    mflat_hbm.at[abufs[b]], rowbufs[b], sg[b]))
            for b in range(_NBUF):
                dg[b].wait()
                dv[b].wait()
                dsc.append(pltpu.async_copy(
                    rowbufs[b], acc.at[vbufs[b]], ss[b], add=True))
            for b in range(_NBUF):
                dsc[b].wait()
            return carry

        lax.fori_loop(0, nsuper, superchunk, 0)
        plsc.subcore_barrier()
        pltpu.sync_copy(
            acc.at[pl.ds(s * rows_per_sub, rows_per_sub)],
            out_hbm.at[pl.ds(s * rows_per_sub, rows_per_sub),
                       pl.ds(c * 128, 128)])

    return agg_kernel(mflat, u_pad, v_pad, zrows)


def kernel(actors_feats, actor_idcs, actor_ctrs, graph_ctrs, graph_feats,
           graph_idcs, graph_turn, graph_control, graph_intersect,
           pre_u, pre_v, suc_u, suc_v, left_u, left_v, right_u, right_v,
           Wa1, ba1, Wa2, ba2, Wm_in, bm_in,
           Wf_ctr, Wf_pre, Wf_suc, Wf_left, Wf_right,
           Wg_ctr, Wg_pre, Wg_suc, Wg_left, Wg_right,
           bf, bg, Wmeta, bmeta,
           Wq_a2m, Wk_a2m, Wv_a2m, Wo_a2m,
           Wq_m2a, Wk_m2a, Wv_m2a, Wo_m2a,
           Wq_a2a, Wk_a2a, Wv_a2a, Wo_a2a,
           Wh1, bh1, Wreg, breg, Wd2, Wcls, bcls):
    npad = _NNP - _NN

    af = jnp.pad(actors_feats.reshape(_NA, 60), ((0, 0), (0, 4)))
    a0 = _mm([af], jnp.pad(Wa1, ((0, 4), (0, 0))), ba1, mode="ln_relu")
    a = _mm([a0], Wa2, ba2, res=a0, mode="ln_relu")

    enc = jnp.pad(jnp.concatenate([graph_ctrs, graph_feats], -1),
                  ((0, npad), (0, 4)))
    m0 = _mm([enc], jnp.pad(Wm_in, ((0, 4), (0, 0))), bm_in, mode="ln_relu")
    metain = jnp.pad(
        jnp.concatenate([graph_turn, graph_control[:, None],
                         graph_intersect[:, None]], -1),
        ((0, npad), (0, 4)))
    meta = _mm([metain], jnp.pad(Wmeta, ((0, 4), (0, 0))), bmeta)

    def pad_edges(u, v):
        e = u.shape[0]
        ep = ((e + _EMULT - 1) // _EMULT) * _EMULT
        return (jnp.pad(u, (0, ep - e)),
                jnp.pad(v, (0, ep - e), constant_values=_NN))

    edge_lists = [pad_edges(pre_u, pre_v), pad_edges(suc_u, suc_v),
                  pad_edges(left_u, left_v), pad_edges(right_u, right_v)]
    zrows = jnp.zeros((_NNP // 16, 128), _F32)

    def edge_aggs(m):
        mflat = jnp.concatenate([m[:, :128], m[:, 128:]], axis=0)
        return [_edge_agg_sc(mflat, u, v, zrows) for (u, v) in edge_lists]

    aggs = edge_aggs(m0)
    Wf = jnp.concatenate([Wf_ctr, Wf_pre, Wf_suc, Wf_left, Wf_right], axis=0)
    m1 = _mm([m0] + aggs, Wf, bf, res=m0, res2=meta, mode="ln_relu")

    gctr = jnp.pad(graph_ctrs, ((0, npad), (0, 0)))

    def attention(x_dst, ctr_d, x_src, ctr_s, n_src_real, Wq, Wk, Wv, Wo):
        nd = x_dst.shape[0]
        ns = x_src.shape[0]
        q = _mm([x_dst], Wq * (1.0 / 16.0))
        kv = _mm([x_src], jnp.concatenate([Wk, Wv], axis=1))
        k, v = kv[:, :_D], kv[:, _D:]
        maskrow = jnp.where(jnp.arange(ns) < n_src_real, 0.0, -1e9)[None, :]
        cd = jnp.pad(ctr_d, ((0, 0), (0, 6)))
        cst = jnp.concatenate(
            [ctr_s.T, -0.1 * jnp.sum(ctr_s ** 2, -1)[None, :],
             maskrow, jnp.zeros((4, ns), _F32)], axis=0)
        ctx = _attn(q, k, v, cd, cst)
        return _mm([ctx], Wo, res=x_dst)

    m2 = attention(m1, gctr, a, actor_ctrs, _NA,
                   Wq_a2m, Wk_a2m, Wv_a2m, Wo_a2m)

    aggs2 = edge_aggs(m2)
    Wg = jnp.concatenate([Wg_ctr, Wg_pre, Wg_suc, Wg_left, Wg_right], axis=0)
    m3 = _mm([m2] + aggs2, Wg, bg, res=m2, mode="ln_relu")

    a2 = attention(a, actor_ctrs, m3, gctr, _NN,
                   Wq_m2a, Wk_m2a, Wv_m2a, Wo_m2a)
    a3 = attention(a2, actor_ctrs, a2, actor_ctrs, _NA,
                   Wq_a2a, Wk_a2a, Wv_a2a, Wo_a2a)

    h = _mm([a3], Wh1, bh1, mode="ln_relu")
    ctr_tiled = jnp.pad(jnp.tile(actor_ctrs, (1, 180)), ((0, 0), (0, 24)))
    regf = _mm([h], jnp.pad(Wreg, ((0, 0), (0, 24))),
               jnp.pad(breg, (0, 24)), res=ctr_tiled)
    reg = regf[:, :360].reshape(_NA, 6, 30, 2)
    dest = reg[:, :, -1, :]
    dd48 = jnp.pad(dest - actor_ctrs[:, None, :],
                   ((0, 0), (0, 0), (0, 6))).reshape(_NA, 48)
    clsout = _cls_head(h, dd48,
                       jnp.pad(Wd2, ((0, 6), (0, 0))),
                       jnp.pad(Wcls, ((0, 0), (0, 127))),
                       jnp.broadcast_to(bcls.reshape(1, 1), (1, 128)))
    cls = clsout.reshape(_NA, 6, 128)[:, :, 0]
    return reg, cls
